# baseline (speedup 1.0000x reference)
import functools

import jax
import jax.numpy as jnp
from jax import lax
from jax.experimental import pallas as pl
from jax.experimental.pallas import tpu as pltpu
from jax.experimental.pallas import tpu_sc as plsc

M = 262144
D = 128
N_IDX = M // 2

NC = 2
NS = 16
PER_TILE = N_IDX // NS
CHUNK = 128
NCHUNK = PER_TILE // CHUNK
SLICE = M // NS
ZBUF = 8192


FIRE = 32


def _flags_body(unmasked_hbm, masked_hbm, fu_hbm, fm_hbm,
                idx_v, ones_v, zero_v, flags_sh, sem_idx, sem_z, sem_sc):
    cid = lax.axis_index("c")
    tid = lax.axis_index("s")

    def _fill_ones(i, _):
        ones_v[pl.ds(i * 16, 16)] = jnp.ones((16,), jnp.float32)
        return 0

    lax.fori_loop(0, CHUNK // 16, _fill_ones, 0)

    def _fill_zero(i, _):
        zero_v[pl.ds(i * 16, 16)] = jnp.zeros((16,), jnp.float32)
        return 0

    lax.fori_loop(0, ZBUF // 16, _fill_zero, 0)

    for j in range(SLICE // ZBUF):
        pltpu.sync_copy(zero_v, flags_sh.at[pl.ds(tid * SLICE + j * ZBUF, ZBUF)])
    plsc.subcore_barrier()

    def _scatter_from(idx_hbm):
        pltpu.sync_copy(idx_hbm.at[tid], idx_v)

        def _round(r, _):
            cps = [pltpu.async_copy(ones_v, flags_sh.at[idx_v.at[r * FIRE + k]],
                                    sem_sc, add=True)
                   for k in range(FIRE)]
            for c in cps:
                c.wait()
            return 0

        lax.fori_loop(0, NCHUNK // FIRE, _round, 0)

    @pl.when(cid == 0)
    def _():
        _scatter_from(unmasked_hbm)

    @pl.when(cid == 1)
    def _():
        _scatter_from(masked_hbm)

    plsc.subcore_barrier()

    @pl.when(cid == 0)
    def _():
        pltpu.sync_copy(flags_sh.at[pl.ds(tid * SLICE, SLICE)],
                        fu_hbm.at[pl.ds(tid * SLICE, SLICE)])

    @pl.when(cid == 1)
    def _():
        pltpu.sync_copy(flags_sh.at[pl.ds(tid * SLICE, SLICE)],
                        fm_hbm.at[pl.ds(tid * SLICE, SLICE)])


def _build_flags(unmasked_idx, masked_idx):
    mesh = plsc.VectorSubcoreMesh(core_axis_name="c", subcore_axis_name="s")
    u3 = unmasked_idx.reshape(NS, NCHUNK, CHUNK)
    m3 = masked_idx.reshape(NS, NCHUNK, CHUNK)
    return pl.kernel(
        _flags_body,
        mesh=mesh,
        out_type=[jax.ShapeDtypeStruct((M,), jnp.float32),
                  jax.ShapeDtypeStruct((M,), jnp.float32)],
        scratch_types=[
            pltpu.VMEM((NCHUNK, CHUNK), jnp.int32),
            pltpu.VMEM((CHUNK,), jnp.float32),
            pltpu.VMEM((ZBUF,), jnp.float32),
            pltpu.VMEM_SHARED((M,), jnp.float32),
            pltpu.SemaphoreType.DMA,
            pltpu.SemaphoreType.DMA,
            pltpu.SemaphoreType.DMA,
        ],
    )(u3, m3)


BLK = 16384
G = BLK // 128


def _select_body(bev_ref, prob_ref, fu_ref, fm_ref, me_ref, out_ref):
    p = prob_ref[...]
    fu = fu_ref[...]
    fm = fm_ref[...]
    a = jnp.where((fu > 0.5) & (fm <= 0.5), p, 0.0)
    b = jnp.where(fm > 0.5, 1.0, 0.0)
    at = a.T
    bt = b.T
    me = me_ref[...]
    for g in range(G):
        ag = jax.lax.slice(at, (0, g), (128, g + 1))
        bg = jax.lax.slice(bt, (0, g), (128, g + 1))
        out_ref[pl.ds(g * 128, 128), :] = (
            ag * bev_ref[pl.ds(g * 128, 128), :] + bg * me)


def _dense_select(bev_queries, prob, mask_embed, fu, fm):
    grid = (M // BLK,)
    return pl.pallas_call(
        _select_body,
        grid=grid,
        in_specs=[
            pl.BlockSpec((BLK, D), lambda i: (i, 0)),
            pl.BlockSpec((G, 128), lambda i: (i, 0)),
            pl.BlockSpec((G, 128), lambda i: (i, 0)),
            pl.BlockSpec((G, 128), lambda i: (i, 0)),
            pl.BlockSpec((1, D), lambda i: (0, 0)),
        ],
        out_specs=pl.BlockSpec((BLK, D), lambda i: (i, 0)),
        out_shape=jax.ShapeDtypeStruct((M, D), jnp.float32),
    )(bev_queries, prob.reshape(M // 128, 128), fu.reshape(M // 128, 128),
      fm.reshape(M // 128, 128), mask_embed)


@jax.jit
def kernel(bev_queries, prob, mask_embed, unmasked_idx, masked_idx):
    fu, fm = _build_flags(unmasked_idx, masked_idx)
    return _dense_select(bev_queries, prob, mask_embed, fu, fm)


# speedup vs baseline: 1.0081x; 1.0081x over previous
import functools

import jax
import jax.numpy as jnp
from jax import lax
from jax.experimental import pallas as pl
from jax.experimental.pallas import tpu as pltpu
from jax.experimental.pallas import tpu_sc as plsc

M = 262144
D = 128
N_IDX = M // 2

NC = 2
NS = 16
PER_TILE = N_IDX // NS
CHUNK = 128
NCHUNK = PER_TILE // CHUNK
SLICE = M // NS
ZBUF = 2048


FIRE = 16


def _flags_body(unmasked_hbm, masked_hbm, fu_hbm, fm_hbm,
                idx_v, ones_v, zero_v, flags_sh, sem_idx, sem_z, sem_sc):
    cid = lax.axis_index("c")
    tid = lax.axis_index("s")

    def _fill_ones(i, _):
        ones_v[pl.ds(i * 16, 16)] = jnp.ones((16,), jnp.float32)
        return 0

    lax.fori_loop(0, CHUNK // 16, _fill_ones, 0)

    def _fill_zero(i, _):
        zero_v[pl.ds(i * 16, 16)] = jnp.zeros((16,), jnp.float32)
        return 0

    lax.fori_loop(0, ZBUF // 16, _fill_zero, 0)

    for j in range(SLICE // ZBUF):
        pltpu.sync_copy(zero_v, flags_sh.at[pl.ds(tid * SLICE + j * ZBUF, ZBUF)])
    plsc.subcore_barrier()

    def _scatter_from(idx_hbm):
        pltpu.sync_copy(idx_hbm.at[tid], idx_v)

        def _round(r, _):
            cps = [pltpu.async_copy(ones_v, flags_sh.at[idx_v.at[r * FIRE + k]],
                                    sem_sc, add=True)
                   for k in range(FIRE)]
            for c in cps:
                c.wait()
            return 0

        lax.fori_loop(0, NCHUNK // FIRE, _round, 0)

    @pl.when(cid == 0)
    def _():
        _scatter_from(unmasked_hbm)

    @pl.when(cid == 1)
    def _():
        _scatter_from(masked_hbm)

    plsc.subcore_barrier()

    @pl.when(cid == 0)
    def _():
        pltpu.sync_copy(flags_sh.at[pl.ds(tid * SLICE, SLICE)],
                        fu_hbm.at[pl.ds(tid * SLICE, SLICE)])

    @pl.when(cid == 1)
    def _():
        pltpu.sync_copy(flags_sh.at[pl.ds(tid * SLICE, SLICE)],
                        fm_hbm.at[pl.ds(tid * SLICE, SLICE)])


def _build_flags(unmasked_idx, masked_idx):
    mesh = plsc.VectorSubcoreMesh(core_axis_name="c", subcore_axis_name="s")
    u3 = unmasked_idx.reshape(NS, NCHUNK, CHUNK)
    m3 = masked_idx.reshape(NS, NCHUNK, CHUNK)
    return pl.kernel(
        _flags_body,
        mesh=mesh,
        out_type=[jax.ShapeDtypeStruct((M,), jnp.float32),
                  jax.ShapeDtypeStruct((M,), jnp.float32)],
        scratch_types=[
            pltpu.VMEM((NCHUNK, CHUNK), jnp.int32),
            pltpu.VMEM((CHUNK,), jnp.float32),
            pltpu.VMEM((ZBUF,), jnp.float32),
            pltpu.VMEM_SHARED((M,), jnp.float32),
            pltpu.SemaphoreType.DMA,
            pltpu.SemaphoreType.DMA,
            pltpu.SemaphoreType.DMA,
        ],
    )(u3, m3)


BLK = 16384
G = BLK // 128


def _select_body(bev_ref, prob_ref, fu_ref, fm_ref, me_ref, out_ref):
    p = prob_ref[...]
    fu = fu_ref[...]
    fm = fm_ref[...]
    a = jnp.where((fu > 0.5) & (fm <= 0.5), p, 0.0)
    b = jnp.where(fm > 0.5, 1.0, 0.0)
    at = a.T
    bt = b.T
    me = me_ref[...]
    for g in range(G):
        ag = jax.lax.slice(at, (0, g), (128, g + 1))
        bg = jax.lax.slice(bt, (0, g), (128, g + 1))
        out_ref[pl.ds(g * 128, 128), :] = (
            ag * bev_ref[pl.ds(g * 128, 128), :] + bg * me)


def _dense_select(bev_queries, prob, mask_embed, fu, fm):
    grid = (M // BLK,)
    return pl.pallas_call(
        _select_body,
        grid=grid,
        in_specs=[
            pl.BlockSpec((BLK, D), lambda i: (i, 0)),
            pl.BlockSpec((G, 128), lambda i: (i, 0)),
            pl.BlockSpec((G, 128), lambda i: (i, 0)),
            pl.BlockSpec((G, 128), lambda i: (i, 0)),
            pl.BlockSpec((1, D), lambda i: (0, 0)),
        ],
        out_specs=pl.BlockSpec((BLK, D), lambda i: (i, 0)),
        out_shape=jax.ShapeDtypeStruct((M, D), jnp.float32),
    )(bev_queries, prob.reshape(M // 128, 128), fu.reshape(M // 128, 128),
      fm.reshape(M // 128, 128), mask_embed)


@jax.jit
def kernel(bev_queries, prob, mask_embed, unmasked_idx, masked_idx):
    fu, fm = _build_flags(unmasked_idx, masked_idx)
    return _dense_select(bev_queries, prob, mask_embed, fu, fm)
